# baseline (speedup 1.0000x reference)
import functools

import jax
import jax.numpy as jnp
from jax import lax
from jax.experimental import pallas as pl
from jax.experimental.pallas import tpu as pltpu
from jax.experimental.pallas import tpu_sc as plsc

_K = 100000
_D = 128
_B = 16384
_NC = 2
_NS = 16
_NW = _NC * _NS
_ROWS = _B // _NW
_G = 16
_NGRP = _ROWS // _G
_TR = (_B * 2 // _D) // _NW


def _xbm_body(params_hbm, feats_hbm, tgt_hbm, fmem_hbm, tmem_hbm,
              outf_hbm, outt_hbm,
              params_v, fbuf, tbuf, mstage, tfstage, tmstage, sem):
    wid = lax.axis_index("s") * _NC + lax.axis_index("c")
    base = wid * _ROWS

    pltpu.sync_copy(params_hbm, params_v)
    pv = params_v[...]
    ws = pv[0]
    os_ = pv[1]
    g0 = os_ + base

    full_f = jnp.logical_and(g0 >= ws, g0 + _ROWS <= ws + _B)
    full_m = jnp.logical_or(g0 + _ROWS <= ws, g0 >= ws + _B)
    src_f = g0 - ws
    case_a = jnp.logical_and(full_f, src_f % 8 == 0)
    case_b = jnp.logical_and(full_m, g0 % 8 == 0)
    case_c = jnp.logical_not(jnp.logical_or(case_a, case_b))

    @pl.when(case_a)
    def _():
        src = pl.multiple_of(src_f, 8)
        pltpu.sync_copy(feats_hbm.at[pl.ds(src, _ROWS)], fbuf)
        pltpu.sync_copy(tgt_hbm.at[pl.ds(src, _ROWS)], tbuf)

    @pl.when(case_b)
    def _():
        src = pl.multiple_of(g0, 8)
        pltpu.sync_copy(fmem_hbm.at[pl.ds(src, _ROWS)], fbuf)
        pltpu.sync_copy(tmem_hbm.at[pl.ds(src, _ROWS)], tbuf)

    @pl.when(case_c)
    def _():
        iota = lax.iota(jnp.int32, 16)

        def group(gi, carry):
            off = gi * _G
            c0 = g0 + off
            gvec = c0 + iota
            fidx = jnp.clip(gvec - ws, 0, _B - 1)

            pltpu.async_copy(feats_hbm.at[fidx],
                             fbuf.at[pl.ds(off, _G)], sem).wait()
            pltpu.async_copy(fmem_hbm.at[gvec], mstage, sem).wait()
            pltpu.async_copy(tgt_hbm.at[fidx], tfstage, sem).wait()
            pltpu.async_copy(tmem_hbm.at[gvec], tmstage, sem).wait()

            def fixrow(r, c2):
                g = c0 + r
                valid = jnp.logical_and(g >= ws, g < ws + _B)

                @pl.when(jnp.logical_not(valid))
                def _():
                    for jc in range(_D // 16):
                        fbuf[off + r, pl.ds(jc * 16, 16)] = (
                            mstage[r, pl.ds(jc * 16, 16)])

                return c2

            lax.fori_loop(jnp.int32(0), jnp.int32(_G), fixrow, jnp.int32(0))

            for w in range(2):
                wl = 16 * w + iota
                rvec = lax.shift_right_logical(wl, jnp.int32(1))
                cvec = lax.bitwise_and(wl, jnp.int32(1))
                gword = c0 + rvec
                validw = jnp.logical_and(gword >= ws, gword < ws + _B)
                tf = plsc.load_gather(tfstage, [rvec, cvec])
                tm = plsc.load_gather(tmstage, [rvec, cvec])
                plsc.store_scatter(tbuf, [off + rvec, cvec],
                                   jnp.where(validw, tf, tm))

            return carry

        lax.fori_loop(jnp.int32(0), jnp.int32(_NGRP), group, jnp.int32(0))

    dst = pl.multiple_of(base, 8)
    pltpu.sync_copy(fbuf, outf_hbm.at[pl.ds(dst, _ROWS)])
    pltpu.sync_copy(tbuf, outt_hbm.at[pl.ds(dst, _ROWS)])


_xbm_call = functools.partial(
    pl.kernel,
    out_type=[
        jax.ShapeDtypeStruct((_B, _D), jnp.float32),
        jax.ShapeDtypeStruct((_B, 2), jnp.int32),
    ],
    mesh=plsc.VectorSubcoreMesh(core_axis_name="c", subcore_axis_name="s"),
    compiler_params=pltpu.CompilerParams(needs_layout_passes=False,
                                         use_tc_tiling_on_sc=False),
    scratch_types=[
        pltpu.VMEM((16,), jnp.int32),
        pltpu.VMEM((_ROWS, _D), jnp.float32),
        pltpu.VMEM((_ROWS, 2), jnp.int32),
        pltpu.VMEM((_G, _D), jnp.float32),
        pltpu.VMEM((_G, 2), jnp.int32),
        pltpu.VMEM((_G, 2), jnp.int32),
        pltpu.SemaphoreType.DMA,
    ],
)(_xbm_body)


def _fast_body(feats_hbm, tgt_hbm, outf_hbm, outt_hbm,
               fbuf, tbuf, sem1, sem2, sem3):
    wid = lax.axis_index("s") * _NC + lax.axis_index("c")
    base = wid * _ROWS
    half = _ROWS // 2
    d0 = pl.multiple_of(base, 8)
    d1 = pl.multiple_of(base + half, 8)
    in0 = pltpu.async_copy(feats_hbm.at[pl.ds(d0, half)],
                           fbuf.at[pl.ds(0, half)], sem1)
    in1 = pltpu.async_copy(feats_hbm.at[pl.ds(d1, half)],
                           fbuf.at[pl.ds(half, half)], sem2)
    tb = pl.multiple_of(wid * _TR, 8)
    int_ = pltpu.async_copy(tgt_hbm.at[pl.ds(tb, _TR)], tbuf, sem3)
    in0.wait()
    out0 = pltpu.async_copy(fbuf.at[pl.ds(0, half)],
                            outf_hbm.at[pl.ds(d0, half)], sem1)
    in1.wait()
    out1 = pltpu.async_copy(fbuf.at[pl.ds(half, half)],
                            outf_hbm.at[pl.ds(d1, half)], sem2)
    int_.wait()
    outt = pltpu.async_copy(tbuf, outt_hbm.at[pl.ds(tb, _TR)], sem3)
    out0.wait()
    out1.wait()
    outt.wait()


_fast_call = functools.partial(
    pl.kernel,
    out_type=[
        jax.ShapeDtypeStruct((_B, _D), jnp.float32),
        jax.ShapeDtypeStruct((_B * 2 // _D, _D), jnp.int32),
    ],
    mesh=plsc.VectorSubcoreMesh(core_axis_name="c", subcore_axis_name="s"),
    scratch_types=[
        pltpu.VMEM((_ROWS, _D), jnp.float32),
        pltpu.VMEM((_TR, _D), jnp.int32),
        pltpu.SemaphoreType.DMA,
        pltpu.SemaphoreType.DMA,
        pltpu.SemaphoreType.DMA,
    ],
)(_fast_body)


def kernel(feats, targets, feats_mem, targets_mem, ptr, total_count):
    q = feats.shape[0]
    wrap = ptr + q > _K
    write_start = jnp.where(wrap, _K - q, ptr)
    write_start = jnp.clip(write_start, 0, _K - q)
    new_ptr = jnp.where(wrap, 0, ptr + q)
    tc = jnp.minimum(total_count + q, _K + 100)
    is_full = tc >= _K
    out_start = jnp.where(is_full, 0, new_ptr - q)
    out_start = jnp.clip(out_start, 0, _K - q)

    def fast(ops):
        f, t, _, _ = ops
        t32 = lax.bitcast_convert_type(t, jnp.int32).reshape(q * 2 // _D, _D)
        of, ot = _fast_call(f, t32)
        return (of, lax.bitcast_convert_type(ot.reshape(q, 1, 2), jnp.int64))

    def general(ops):
        f, t, fm, tm = ops
        params = jnp.stack([write_start, out_start]).astype(jnp.int32)
        params = jnp.pad(params, (0, 14))
        tgt32 = lax.bitcast_convert_type(t, jnp.int32).reshape(q, 2)
        tmem32 = lax.bitcast_convert_type(tm, jnp.int32).reshape(_K, 2)
        out_feats, outt = _xbm_call(params, f, tgt32, fm, tmem32)
        out_targets = lax.bitcast_convert_type(
            outt.reshape(q, 1, 2), jnp.int64)
        return (out_feats, out_targets)

    return lax.cond(write_start == out_start, fast, general,
                    (feats, targets, feats_mem, targets_mem))
